# baseline (speedup 1.0000x reference)
import jax
import jax.numpy as jnp
from jax.experimental import pallas as pl
from jax.experimental.pallas import tpu as pltpu

_EPS = 1e-5


def _toeplitz_cw(w_oihw, cin, w, kw, wout, cout):
    taps = jnp.transpose(w_oihw[:, :, 0, :], (1, 2, 0))
    pad = wout - 1
    table = jnp.concatenate(
        [jnp.zeros((cin, pad, cout), taps.dtype), taps,
         jnp.zeros((cin, pad, cout), taps.dtype)], axis=1)
    wi = jnp.arange(w)[:, None]
    wo = jnp.arange(wout)[None, :]
    idx = wi - wo + pad
    t4 = table[:, idx, :]
    return (jnp.transpose(t4, (0, 1, 3, 2))
            .reshape(cin * w, cout * wout).astype(jnp.bfloat16))


def _make_p1(bn, cin, h, w, wc_out):
    def _p1(x_ref, w_ref, y_ref, sum_ref, ssq_ref):
        @pl.when(pl.program_id(1) == 0)
        def _():
            sum_ref[...] = jnp.zeros_like(sum_ref)
            ssq_ref[...] = jnp.zeros_like(ssq_ref)

        xb = x_ref[...].astype(jnp.bfloat16)
        lhs = (xb.reshape(bn, cin, h, w).transpose(0, 2, 1, 3)
               .reshape(bn * h, cin * w))
        y = jnp.dot(lhs, w_ref[...], preferred_element_type=jnp.float32)
        y_ref[...] = y.astype(jnp.bfloat16)
        sum_ref[0] += jnp.sum(y, axis=0, keepdims=True)
        ssq_ref[0] += jnp.sum(y * y, axis=0, keepdims=True)
    return _p1


def _p2(y_ref, scale_ref, shift_ref, o_ref):
    y = y_ref[...].astype(jnp.float32)
    o_ref[...] = jnp.maximum(y * scale_ref[...] + shift_ref[...], 0.0)


def kernel(x_nchw, w_oihw, bias, gamma, beta):
    del bias
    n, cin, h, w = x_nchw.shape
    cout, cin_w, kh, kw = w_oihw.shape
    assert kh == 1 and cin_w == cin and w >= kw
    wout = w - kw + 1
    m = n * h
    wc_in = w * cin
    wc_out = wout * cout

    x2 = x_nchw.reshape(n * cin, h * w)
    w_toe = _toeplitz_cw(w_oihw, cin, w, kw, wout, cout)

    bn = 8
    while n % (2 * bn) != 0 and bn > 1:
        bn //= 2
    cores = 2 if n % (2 * bn) == 0 else 1
    steps = n // (bn * cores)

    y2d, lane_sum, lane_ssq = pl.pallas_call(
        _make_p1(bn, cin, h, w, wc_out),
        out_shape=(jax.ShapeDtypeStruct((m, wc_out), jnp.bfloat16),
                   jax.ShapeDtypeStruct((cores, 1, wc_out), jnp.float32),
                   jax.ShapeDtypeStruct((cores, 1, wc_out), jnp.float32)),
        grid=(cores, steps),
        in_specs=[pl.BlockSpec((bn * cin, h * w), lambda c, i, t=steps: (c * t + i, 0)),
                  pl.BlockSpec((wc_in, wc_out), lambda c, i: (0, 0))],
        out_specs=(pl.BlockSpec((bn * h, wc_out), lambda c, i, t=steps: (c * t + i, 0)),
                   pl.BlockSpec((1, 1, wc_out), lambda c, i: (c, 0, 0)),
                   pl.BlockSpec((1, 1, wc_out), lambda c, i: (c, 0, 0))),
        compiler_params=pltpu.CompilerParams(
            dimension_semantics=("parallel", "arbitrary")),
        cost_estimate=pl.CostEstimate(
            flops=2 * m * wc_in * wc_out, transcendentals=0,
            bytes_accessed=4 * m * wc_in + 2 * m * wc_out + 2 * wc_in * wc_out),
    )(x2, w_toe)

    cnt = float(m * wout)
    s = jnp.sum(lane_sum.reshape(cores, cout, wout), axis=(0, 2))
    sq = jnp.sum(lane_ssq.reshape(cores, cout, wout), axis=(0, 2))
    mean = s / cnt
    var = jnp.maximum(sq / cnt - mean * mean, 0.0)
    inv_std = jax.lax.rsqrt(var + _EPS)
    scale_c = gamma.astype(jnp.float32) * inv_std
    shift_c = beta.astype(jnp.float32) - mean * scale_c
    scale_row = jnp.repeat(scale_c, wout).reshape(1, wc_out)
    shift_row = jnp.repeat(shift_c, wout).reshape(1, wc_out)

    out2d = pl.pallas_call(
        _p2,
        out_shape=jax.ShapeDtypeStruct((m, wc_out), jnp.float32),
        grid=(cores * steps,),
        in_specs=[pl.BlockSpec((bn * h, wc_out), lambda i: (i, 0)),
                  pl.BlockSpec((1, wc_out), lambda i: (0, 0)),
                  pl.BlockSpec((1, wc_out), lambda i: (0, 0))],
        out_specs=pl.BlockSpec((bn * h, wc_out), lambda i: (i, 0)),
        compiler_params=pltpu.CompilerParams(
            dimension_semantics=("parallel",)),
        cost_estimate=pl.CostEstimate(
            flops=4 * m * wc_out, transcendentals=0,
            bytes_accessed=2 * m * wc_out + 4 * m * wc_out),
    )(y2d, scale_row, shift_row)

    out = out2d.reshape(n, h, cout, wout)
    return jnp.transpose(out, (0, 2, 1, 3))
